# baseline (speedup 1.0000x reference)
import functools

import jax
import jax.numpy as jnp
from jax import lax
from jax.experimental import pallas as pl
from jax.experimental.pallas import tpu as pltpu
from jax.experimental.pallas import tpu_sc as plsc

N = 10000
E = 640000
H = 128
IN_PAD = 8

NC = 2
NS = 16
NW = NC * NS
NP = 10240
RPS = NP // NS
TRASH = NP - 1

B = 128
CH = 160
E2 = NW * CH * B
IBR = 8
NBLK = CH // IBR

RB = 1024
GRID = NP // RB


def _sc_mesh():
    return plsc.VectorSubcoreMesh(core_axis_name="c", subcore_axis_name="s")


def _deg_kernel(dst3, zeros1):
    @functools.partial(
        pl.kernel,
        out_type=jax.ShapeDtypeStruct((NC * NP,), jnp.float32),
        mesh=_sc_mesh(),
        scratch_types=[
            pltpu.VMEM((CH, B), jnp.int32),
            pltpu.VMEM((B,), jnp.float32),
            pltpu.VMEM_SHARED((NP,), jnp.float32),
            pltpu.SemaphoreType.DMA,
        ],
    )
    def k(dst_hbm, z_hbm, degp_hbm, dst_all, ones_v, acc, dsem):
        c = lax.axis_index("c")
        s = lax.axis_index("s")
        wid = c * NS + s
        r0 = s * RPS
        for i in range(B // 16):
            ones_v[pl.ds(i * 16, 16)] = jnp.ones((16,), jnp.float32)
        pltpu.sync_copy(dst_hbm.at[wid], dst_all)
        pltpu.sync_copy(z_hbm.at[pl.ds(r0, RPS)], acc.at[pl.ds(r0, RPS)])
        plsc.subcore_barrier()

        @pl.loop(0, CH)
        def _fire(j):
            pltpu.async_copy(ones_v, acc.at[dst_all.at[j]], dsem, add=True)

        @pl.loop(0, CH)
        def _drain(j):
            pltpu.make_async_copy(ones_v, acc.at[dst_all.at[0]], dsem).wait()

        plsc.subcore_barrier()
        pltpu.sync_copy(acc.at[pl.ds(r0, RPS)],
                        degp_hbm.at[pl.ds(c * NP + r0, RPS)])

    return k(dst3, zeros1)


def _agg_kernel(y, src3, dst3, zeros2):
    @functools.partial(
        pl.kernel,
        out_type=jax.ShapeDtypeStruct((NC * NP, H), jnp.float32),
        mesh=_sc_mesh(),
        scratch_types=[
            pltpu.VMEM((2, IBR, B), jnp.int32),
            pltpu.VMEM((2, IBR, B), jnp.int32),
            pltpu.VMEM((2, B, H), jnp.float32),
            pltpu.VMEM_SHARED((NP, H), jnp.float32),
            pltpu.SemaphoreType.DMA((2,)),
            pltpu.SemaphoreType.DMA((2,)),
            pltpu.SemaphoreType.DMA((2,)),
        ],
    )
    def k(y_hbm, src_hbm, dst_hbm, z_hbm, part_hbm,
          sblk, dblk, rows, acc, bsem, gsem, ssem):
        c = lax.axis_index("c")
        s = lax.axis_index("s")
        wid = c * NS + s
        r0 = s * RPS
        pltpu.sync_copy(z_hbm.at[pl.ds(r0, RPS)], acc.at[pl.ds(r0, RPS)])

        def fire_blk(kb, bs):
            pltpu.async_copy(src_hbm.at[wid].at[pl.ds(kb * IBR, IBR)],
                             sblk.at[bs], bsem.at[bs])
            pltpu.async_copy(dst_hbm.at[wid].at[pl.ds(kb * IBR, IBR)],
                             dblk.at[bs], bsem.at[bs])

        def wait_blk(bs):
            pltpu.make_async_copy(src_hbm.at[0].at[pl.ds(0, IBR)],
                                  sblk.at[bs], bsem.at[bs]).wait()
            pltpu.make_async_copy(dst_hbm.at[0].at[pl.ds(0, IBR)],
                                  dblk.at[bs], bsem.at[bs]).wait()

        def fire_gather(bs, u, b2):
            pltpu.async_copy(y_hbm.at[sblk.at[bs, u]], rows.at[b2],
                             gsem.at[b2])

        def wait_gather(b2):
            pltpu.make_async_copy(y_hbm.at[sblk.at[0, 0]], rows.at[b2],
                                  gsem.at[b2]).wait()

        def fire_scatter(bs, u, b2):
            pltpu.async_copy(rows.at[b2], acc.at[dblk.at[bs, u]],
                             ssem.at[b2], add=True)

        def wait_scatter(b2):
            pltpu.make_async_copy(rows.at[0], acc.at[dblk.at[0, 0]],
                                  ssem.at[b2]).wait()

        fire_blk(0, 0)
        fire_blk(1, 1)
        plsc.subcore_barrier()

        @pl.loop(0, NBLK // 2)
        def _pair(t):
            for half in range(2):
                bs = half
                wait_blk(bs)
                for u in range(IBR):
                    b2 = u % 2
                    if half == 0 and u < 2:
                        @pl.when(t > 0)
                        def _():
                            wait_scatter(b2)
                    else:
                        wait_scatter(b2)
                    if u == 2:
                        m = t * 2 + half
                        if half == 0:
                            @pl.when(t > 0)
                            def _():
                                fire_blk(m + 1, 1 - bs)
                        else:
                            @pl.when(t < NBLK // 2 - 1)
                            def _():
                                fire_blk(m + 1, 1 - bs)
                    fire_gather(bs, u, b2)
                    wait_gather(b2)
                    fire_scatter(bs, u, b2)

        wait_scatter(0)
        wait_scatter(1)
        plsc.subcore_barrier()
        pltpu.sync_copy(acc.at[pl.ds(r0, RPS)],
                        part_hbm.at[pl.ds(c * NP + r0, RPS)])

    return k(y, src3, dst3, zeros2)


def _prep_kernel(xp, w1p, deg0, deg1):
    def body(x_ref, w_ref, d0_ref, d1_ref, y_ref, dinv_ref):
        deg = d0_ref[...] + d1_ref[...] + 1.0
        dinv = lax.rsqrt(deg)
        xw = jnp.dot(x_ref[...], w_ref[...],
                     preferred_element_type=jnp.float32)
        y_ref[...] = xw * dinv
        dinv_ref[...] = dinv

    return pl.pallas_call(
        body,
        grid=(GRID,),
        in_specs=[
            pl.BlockSpec((RB, IN_PAD), lambda i: (i, 0)),
            pl.BlockSpec((IN_PAD, H), lambda i: (0, 0)),
            pl.BlockSpec((RB, 1), lambda i: (i, 0)),
            pl.BlockSpec((RB, 1), lambda i: (i, 0)),
        ],
        out_specs=[
            pl.BlockSpec((RB, H), lambda i: (i, 0)),
            pl.BlockSpec((RB, 1), lambda i: (i, 0)),
        ],
        out_shape=[
            jax.ShapeDtypeStruct((NP, H), jnp.float32),
            jax.ShapeDtypeStruct((NP, 1), jnp.float32),
        ],
    )(xp, w1p, deg0, deg1)


def _combine_kernel(p0, p1, y, dinv, b, w_next):
    def body(p0_ref, p1_ref, y_ref, dinv_ref, b_ref, w_ref, out_ref):
        dinv = dinv_ref[...]
        h = (p0_ref[...] + p1_ref[...] + y_ref[...]) * dinv + b_ref[...]
        h = jnp.maximum(h, 0.0)
        out_ref[...] = jnp.dot(h, w_ref[...],
                               preferred_element_type=jnp.float32) * dinv

    return pl.pallas_call(
        body,
        grid=(GRID,),
        in_specs=[
            pl.BlockSpec((RB, H), lambda i: (i, 0)),
            pl.BlockSpec((RB, H), lambda i: (i, 0)),
            pl.BlockSpec((RB, H), lambda i: (i, 0)),
            pl.BlockSpec((RB, 1), lambda i: (i, 0)),
            pl.BlockSpec((1, H), lambda i: (0, 0)),
            pl.BlockSpec((H, H), lambda i: (0, 0)),
        ],
        out_specs=pl.BlockSpec((RB, H), lambda i: (i, 0)),
        out_shape=jax.ShapeDtypeStruct((NP, H), jnp.float32),
    )(p0, p1, y, dinv, b, w_next)


def _final_kernel(p0, p1, y, dinv, b3, q1, qb1, q2, qb2, q3, qb3):
    def body(p0_ref, p1_ref, y_ref, dinv_ref, b_ref,
             q1_ref, qb1_ref, q2_ref, qb2_ref, q3_ref, qb3_ref, out_ref):
        dinv = dinv_ref[...]
        h = (p0_ref[...] + p1_ref[...] + y_ref[...]) * dinv + b_ref[...]
        h = jnp.maximum(h, 0.0)
        t = jnp.dot(h, q1_ref[...], preferred_element_type=jnp.float32)
        t = t + qb1_ref[...]
        t = jnp.where(t >= 0.0, t, 0.01 * t)
        t = jnp.dot(t, q2_ref[...], preferred_element_type=jnp.float32)
        t = t + qb2_ref[...]
        t = jnp.where(t >= 0.0, t, 0.01 * t)
        t = jnp.dot(t, q3_ref[...], preferred_element_type=jnp.float32)
        out_ref[...] = t + qb3_ref[...]

    return pl.pallas_call(
        body,
        grid=(GRID,),
        in_specs=[
            pl.BlockSpec((RB, H), lambda i: (i, 0)),
            pl.BlockSpec((RB, H), lambda i: (i, 0)),
            pl.BlockSpec((RB, H), lambda i: (i, 0)),
            pl.BlockSpec((RB, 1), lambda i: (i, 0)),
            pl.BlockSpec((1, H), lambda i: (0, 0)),
            pl.BlockSpec((H, H), lambda i: (0, 0)),
            pl.BlockSpec((1, H), lambda i: (0, 0)),
            pl.BlockSpec((H, H), lambda i: (0, 0)),
            pl.BlockSpec((1, H), lambda i: (0, 0)),
            pl.BlockSpec((H, 1), lambda i: (0, 0)),
            pl.BlockSpec((1, 1), lambda i: (0, 0)),
        ],
        out_specs=pl.BlockSpec((RB, 1), lambda i: (i, 0)),
        out_shape=jax.ShapeDtypeStruct((NP, 1), jnp.float32),
    )(p0, p1, y, dinv, b3, q1, qb1, q2, qb2, q3, qb3)


def kernel(x, edge_index, W1, b1, W2, b2, W3, b3, Q1, qb1, Q2, qb2, Q3, qb3):
    pad = jnp.full((E2 - E,), TRASH, jnp.int32)
    src3 = jnp.concatenate([edge_index[0], pad]).reshape(NW, CH, B)
    dst3 = jnp.concatenate([edge_index[1], pad]).reshape(NW, CH, B)

    xp = jnp.zeros((NP, IN_PAD), jnp.float32).at[:N, :x.shape[1]].set(x)
    w1p = jnp.zeros((IN_PAD, H), jnp.float32).at[:W1.shape[0], :].set(W1)
    zeros1 = jnp.zeros((NP,), jnp.float32)
    zeros2 = jnp.zeros((NP, H), jnp.float32)

    degp = _deg_kernel(dst3, zeros1)
    deg0 = degp[:NP].reshape(NP, 1)
    deg1 = degp[NP:].reshape(NP, 1)

    y1, dinv = _prep_kernel(xp, w1p, deg0, deg1)

    p = _agg_kernel(y1, src3, dst3, zeros2)
    y2 = _combine_kernel(p[:NP], p[NP:], y1, dinv, b1.reshape(1, H), W2)

    p = _agg_kernel(y2, src3, dst3, zeros2)
    y3 = _combine_kernel(p[:NP], p[NP:], y2, dinv, b2.reshape(1, H), W3)

    p = _agg_kernel(y3, src3, dst3, zeros2)
    out = _final_kernel(p[:NP], p[NP:], y3, dinv, b3.reshape(1, H),
                        Q1, qb1.reshape(1, H), Q2, qb2.reshape(1, H),
                        Q3, qb3.reshape(1, 1))
    return out[:N]


# speedup vs baseline: 3.5723x; 3.5723x over previous
import functools

import jax
import jax.numpy as jnp
from jax import lax
from jax.experimental import pallas as pl
from jax.experimental.pallas import tpu as pltpu
from jax.experimental.pallas import tpu_sc as plsc

N = 10000
E = 640000
H = 128
IN_PAD = 8

NC = 2
NS = 16
NW = NC * NS
NP = 10240
RPS = NP // NS

B = 128
CH = 160
E2 = NW * CH * B
IBR = 8
NBLK = CH // IBR

RB = 1024
GRID = NP // RB


def _sc_mesh():
    return plsc.VectorSubcoreMesh(core_axis_name="c", subcore_axis_name="s")


def _deg_kernel(dst3, zeros1):
    @functools.partial(
        pl.kernel,
        out_type=jax.ShapeDtypeStruct((NC * NP,), jnp.float32),
        mesh=_sc_mesh(),
        scratch_types=[
            pltpu.VMEM((CH, B), jnp.int32),
            pltpu.VMEM((B,), jnp.float32),
            pltpu.VMEM_SHARED((NP,), jnp.float32),
            pltpu.SemaphoreType.DMA,
        ],
    )
    def k(dst_hbm, z_hbm, degp_hbm, dst_all, ones_v, acc, dsem):
        c = lax.axis_index("c")
        s = lax.axis_index("s")
        wid = c * NS + s
        r0 = s * RPS
        for i in range(B // 16):
            ones_v[pl.ds(i * 16, 16)] = jnp.ones((16,), jnp.float32)
        pltpu.sync_copy(dst_hbm.at[wid], dst_all)
        pltpu.sync_copy(z_hbm.at[pl.ds(r0, RPS)], acc.at[pl.ds(r0, RPS)])
        plsc.subcore_barrier()

        @pl.loop(0, CH)
        def _fire(j):
            pltpu.async_copy(ones_v, acc.at[dst_all.at[j]], dsem, add=True)

        @pl.loop(0, CH)
        def _drain(j):
            pltpu.make_async_copy(ones_v, acc.at[dst_all.at[0]], dsem).wait()

        plsc.subcore_barrier()
        pltpu.sync_copy(acc.at[pl.ds(r0, RPS)],
                        degp_hbm.at[pl.ds(c * NP + r0, RPS)])

    return k(dst3, zeros1)


def _agg_kernel(y, src3, dst3, zeros2):
    @functools.partial(
        pl.kernel,
        out_type=jax.ShapeDtypeStruct((NC * NP, H), jnp.float32),
        mesh=_sc_mesh(),
        scratch_types=[
            pltpu.VMEM((2, IBR, B), jnp.int32),
            pltpu.VMEM((2, IBR, B), jnp.int32),
            pltpu.VMEM((2, B, H), jnp.float32),
            pltpu.VMEM_SHARED((NP, H), jnp.float32),
            pltpu.SemaphoreType.DMA((2,)),
            pltpu.SemaphoreType.DMA((2,)),
            pltpu.SemaphoreType.DMA((2,)),
        ],
    )
    def k(y_hbm, src_hbm, dst_hbm, z_hbm, part_hbm,
          sblk, dblk, rows, acc, bsem, gsem, ssem):
        c = lax.axis_index("c")
        s = lax.axis_index("s")
        wid = c * NS + s
        r0 = s * RPS
        pltpu.sync_copy(z_hbm.at[pl.ds(r0, RPS)], acc.at[pl.ds(r0, RPS)])

        def fire_blk(kb, bs):
            pltpu.async_copy(src_hbm.at[wid].at[pl.ds(kb * IBR, IBR)],
                             sblk.at[bs], bsem.at[bs])
            pltpu.async_copy(dst_hbm.at[wid].at[pl.ds(kb * IBR, IBR)],
                             dblk.at[bs], bsem.at[bs])

        def wait_blk(bs):
            pltpu.make_async_copy(src_hbm.at[0].at[pl.ds(0, IBR)],
                                  sblk.at[bs], bsem.at[bs]).wait()
            pltpu.make_async_copy(dst_hbm.at[0].at[pl.ds(0, IBR)],
                                  dblk.at[bs], bsem.at[bs]).wait()

        def fire_gather(bs, u, b2):
            pltpu.async_copy(y_hbm.at[sblk.at[bs, u]], rows.at[b2],
                             gsem.at[b2])

        def wait_gather(b2):
            pltpu.make_async_copy(y_hbm.at[sblk.at[0, 0]], rows.at[b2],
                                  gsem.at[b2]).wait()

        def fire_scatter(bs, u, b2):
            pltpu.async_copy(rows.at[b2], acc.at[dblk.at[bs, u]],
                             ssem.at[b2], add=True)

        def wait_scatter(b2):
            pltpu.make_async_copy(rows.at[0], acc.at[dblk.at[0, 0]],
                                  ssem.at[b2]).wait()

        fire_blk(0, 0)
        fire_blk(1, 1)
        plsc.subcore_barrier()

        @pl.loop(0, NBLK // 2)
        def _pair(t):
            for half in range(2):
                bs = half
                wait_blk(bs)
                for u in range(IBR):
                    b2 = u % 2
                    if half == 0 and u < 2:
                        @pl.when(t > 0)
                        def _():
                            wait_scatter(b2)
                    else:
                        wait_scatter(b2)
                    if u == 2:
                        m = t * 2 + half
                        if half == 0:
                            @pl.when(t > 0)
                            def _():
                                fire_blk(m + 1, 1 - bs)
                        else:
                            @pl.when(t < NBLK // 2 - 1)
                            def _():
                                fire_blk(m + 1, 1 - bs)
                    fire_gather(bs, u, b2)
                    wait_gather(b2)
                    fire_scatter(bs, u, b2)

        wait_scatter(0)
        wait_scatter(1)
        plsc.subcore_barrier()
        pltpu.sync_copy(acc.at[pl.ds(r0, RPS)],
                        part_hbm.at[pl.ds(c * NP + r0, RPS)])

    return k(y, src3, dst3, zeros2)


def _prep_kernel(xp, w1p, deg0, deg1):
    def body(x_ref, w_ref, d0_ref, d1_ref, y_ref, dinv_ref):
        deg = d0_ref[...] + d1_ref[...] + 1.0
        dinv = lax.rsqrt(deg)
        xw = jnp.dot(x_ref[...], w_ref[...],
                     preferred_element_type=jnp.float32)
        y_ref[...] = xw * dinv
        dinv_ref[...] = dinv

    return pl.pallas_call(
        body,
        grid=(GRID,),
        in_specs=[
            pl.BlockSpec((RB, IN_PAD), lambda i: (i, 0)),
            pl.BlockSpec((IN_PAD, H), lambda i: (0, 0)),
            pl.BlockSpec((RB, 1), lambda i: (i, 0)),
            pl.BlockSpec((RB, 1), lambda i: (i, 0)),
        ],
        out_specs=[
            pl.BlockSpec((RB, H), lambda i: (i, 0)),
            pl.BlockSpec((RB, 1), lambda i: (i, 0)),
        ],
        out_shape=[
            jax.ShapeDtypeStruct((NP, H), jnp.float32),
            jax.ShapeDtypeStruct((NP, 1), jnp.float32),
        ],
    )(xp, w1p, deg0, deg1)


def _combine_kernel(p0, p1, y, dinv, b, w_next):
    def body(p0_ref, p1_ref, y_ref, dinv_ref, b_ref, w_ref, out_ref):
        dinv = dinv_ref[...]
        h = (p0_ref[...] + p1_ref[...] + y_ref[...]) * dinv + b_ref[...]
        h = jnp.maximum(h, 0.0)
        out_ref[...] = jnp.dot(h, w_ref[...],
                               preferred_element_type=jnp.float32) * dinv

    return pl.pallas_call(
        body,
        grid=(GRID,),
        in_specs=[
            pl.BlockSpec((RB, H), lambda i: (i, 0)),
            pl.BlockSpec((RB, H), lambda i: (i, 0)),
            pl.BlockSpec((RB, H), lambda i: (i, 0)),
            pl.BlockSpec((RB, 1), lambda i: (i, 0)),
            pl.BlockSpec((1, H), lambda i: (0, 0)),
            pl.BlockSpec((H, H), lambda i: (0, 0)),
        ],
        out_specs=pl.BlockSpec((RB, H), lambda i: (i, 0)),
        out_shape=jax.ShapeDtypeStruct((NP, H), jnp.float32),
    )(p0, p1, y, dinv, b, w_next)


def _final_kernel(p0, p1, y, dinv, b3, q1, qb1, q2, qb2, q3, qb3):
    def body(p0_ref, p1_ref, y_ref, dinv_ref, b_ref,
             q1_ref, qb1_ref, q2_ref, qb2_ref, q3_ref, qb3_ref, out_ref):
        dinv = dinv_ref[...]
        h = (p0_ref[...] + p1_ref[...] + y_ref[...]) * dinv + b_ref[...]
        h = jnp.maximum(h, 0.0)
        t = jnp.dot(h, q1_ref[...], preferred_element_type=jnp.float32)
        t = t + qb1_ref[...]
        t = jnp.where(t >= 0.0, t, 0.01 * t)
        t = jnp.dot(t, q2_ref[...], preferred_element_type=jnp.float32)
        t = t + qb2_ref[...]
        t = jnp.where(t >= 0.0, t, 0.01 * t)
        t = jnp.dot(t, q3_ref[...], preferred_element_type=jnp.float32)
        out_ref[...] = t + qb3_ref[...]

    return pl.pallas_call(
        body,
        grid=(GRID,),
        in_specs=[
            pl.BlockSpec((RB, H), lambda i: (i, 0)),
            pl.BlockSpec((RB, H), lambda i: (i, 0)),
            pl.BlockSpec((RB, H), lambda i: (i, 0)),
            pl.BlockSpec((RB, 1), lambda i: (i, 0)),
            pl.BlockSpec((1, H), lambda i: (0, 0)),
            pl.BlockSpec((H, H), lambda i: (0, 0)),
            pl.BlockSpec((1, H), lambda i: (0, 0)),
            pl.BlockSpec((H, H), lambda i: (0, 0)),
            pl.BlockSpec((1, H), lambda i: (0, 0)),
            pl.BlockSpec((H, 1), lambda i: (0, 0)),
            pl.BlockSpec((1, 1), lambda i: (0, 0)),
        ],
        out_specs=pl.BlockSpec((RB, 1), lambda i: (i, 0)),
        out_shape=jax.ShapeDtypeStruct((NP, 1), jnp.float32),
    )(p0, p1, y, dinv, b3, q1, qb1, q2, qb2, q3, qb3)


def kernel(x, edge_index, W1, b1, W2, b2, W3, b3, Q1, qb1, Q2, qb2, Q3, qb3):
    pad = N + (jnp.arange(E2 - E, dtype=jnp.int32) % (NP - N))
    src3 = jnp.concatenate([edge_index[0], pad]).reshape(NW, CH, B)
    dst3 = jnp.concatenate([edge_index[1], pad]).reshape(NW, CH, B)

    xp = jnp.zeros((NP, IN_PAD), jnp.float32).at[:N, :x.shape[1]].set(x)
    w1p = jnp.zeros((IN_PAD, H), jnp.float32).at[:W1.shape[0], :].set(W1)
    zeros1 = jnp.zeros((NP,), jnp.float32)
    zeros2 = jnp.zeros((NP, H), jnp.float32)

    degp = _deg_kernel(dst3, zeros1)
    deg0 = degp[:NP].reshape(NP, 1)
    deg1 = degp[NP:].reshape(NP, 1)

    y1, dinv = _prep_kernel(xp, w1p, deg0, deg1)

    p = _agg_kernel(y1, src3, dst3, zeros2)
    y2 = _combine_kernel(p[:NP], p[NP:], y1, dinv, b1.reshape(1, H), W2)

    p = _agg_kernel(y2, src3, dst3, zeros2)
    y3 = _combine_kernel(p[:NP], p[NP:], y2, dinv, b2.reshape(1, H), W3)

    p = _agg_kernel(y3, src3, dst3, zeros2)
    out = _final_kernel(p[:NP], p[NP:], y3, dinv, b3.reshape(1, H),
                        Q1, qb1.reshape(1, H), Q2, qb2.reshape(1, H),
                        Q3, qb3.reshape(1, 1))
    return out[:N]


# speedup vs baseline: 4.3720x; 1.2238x over previous
import functools

import jax
import jax.numpy as jnp
from jax import lax
from jax.experimental import pallas as pl
from jax.experimental.pallas import tpu as pltpu
from jax.experimental.pallas import tpu_sc as plsc

N = 10000
E = 640000
H = 128
IN_PAD = 8

NC = 2
NS = 16
NW = NC * NS
NP = 10240
RPS = NP // NS

B = 128
CH = 160
E2 = NW * CH * B
IBR = 8
NBLK = CH // IBR
BSUB = 64
SUBS = IBR * 2

RB = 1024
GRID = NP // RB


def _sc_mesh():
    return plsc.VectorSubcoreMesh(core_axis_name="c", subcore_axis_name="s")


def _deg_kernel(dst3, zeros1):
    @functools.partial(
        pl.kernel,
        out_type=jax.ShapeDtypeStruct((NC * NP,), jnp.float32),
        mesh=_sc_mesh(),
        scratch_types=[
            pltpu.VMEM((CH, B), jnp.int32),
            pltpu.VMEM((B,), jnp.float32),
            pltpu.VMEM_SHARED((NP,), jnp.float32),
            pltpu.SemaphoreType.DMA,
        ],
    )
    def k(dst_hbm, z_hbm, degp_hbm, dst_all, ones_v, acc, dsem):
        c = lax.axis_index("c")
        s = lax.axis_index("s")
        wid = c * NS + s
        r0 = s * RPS
        for i in range(B // 16):
            ones_v[pl.ds(i * 16, 16)] = jnp.ones((16,), jnp.float32)
        pltpu.sync_copy(dst_hbm.at[wid], dst_all)
        pltpu.sync_copy(z_hbm.at[pl.ds(r0, RPS)], acc.at[pl.ds(r0, RPS)])
        plsc.subcore_barrier()

        @pl.loop(0, CH)
        def _fire(j):
            pltpu.async_copy(ones_v, acc.at[dst_all.at[j]], dsem, add=True)

        @pl.loop(0, CH)
        def _drain(j):
            pltpu.make_async_copy(ones_v, acc.at[dst_all.at[0]], dsem).wait()

        plsc.subcore_barrier()
        pltpu.sync_copy(acc.at[pl.ds(r0, RPS)],
                        degp_hbm.at[pl.ds(c * NP + r0, RPS)])

    return k(dst3, zeros1)


def _agg_kernel(y, src3, dst3, zeros2):
    @functools.partial(
        pl.kernel,
        out_type=jax.ShapeDtypeStruct((NC * NP, H), jnp.float32),
        mesh=_sc_mesh(),
        scratch_types=[
            pltpu.VMEM((2, IBR, B), jnp.int32),
            pltpu.VMEM((2, IBR, B), jnp.int32),
            pltpu.VMEM((4, BSUB, H), jnp.float32),
            pltpu.VMEM_SHARED((NP, H), jnp.float32),
            pltpu.SemaphoreType.DMA((2,)),
            pltpu.SemaphoreType.DMA((4,)),
            pltpu.SemaphoreType.DMA((4,)),
        ],
    )
    def k(y_hbm, src_hbm, dst_hbm, z_hbm, part_hbm,
          sblk, dblk, rows, acc, bsem, gsem, ssem):
        c = lax.axis_index("c")
        s = lax.axis_index("s")
        wid = c * NS + s
        r0 = s * RPS
        pltpu.sync_copy(z_hbm.at[pl.ds(r0, RPS)], acc.at[pl.ds(r0, RPS)])

        def fire_blk(kb, bs):
            pltpu.async_copy(src_hbm.at[wid].at[pl.ds(kb * IBR, IBR)],
                             sblk.at[bs], bsem.at[bs])
            pltpu.async_copy(dst_hbm.at[wid].at[pl.ds(kb * IBR, IBR)],
                             dblk.at[bs], bsem.at[bs])

        def wait_blk(bs):
            pltpu.make_async_copy(src_hbm.at[0].at[pl.ds(0, IBR)],
                                  sblk.at[bs], bsem.at[bs]).wait()
            pltpu.make_async_copy(dst_hbm.at[0].at[pl.ds(0, IBR)],
                                  dblk.at[bs], bsem.at[bs]).wait()

        def idx(blk, bs, jl):
            return blk.at[bs, jl // 2, pl.ds((jl % 2) * BSUB, BSUB)]

        def fire_gather(bs, jl, q):
            pltpu.async_copy(y_hbm.at[idx(sblk, bs, jl)], rows.at[q],
                             gsem.at[q])

        def wait_gather(q):
            pltpu.make_async_copy(y_hbm.at[idx(sblk, 0, 0)], rows.at[q],
                                  gsem.at[q]).wait()

        def fire_scatter(bs, jl, q):
            pltpu.async_copy(rows.at[q], acc.at[idx(dblk, bs, jl)],
                             ssem.at[q], add=True)

        def wait_scatter(q):
            pltpu.make_async_copy(rows.at[0], acc.at[idx(dblk, 0, 0)],
                                  ssem.at[q]).wait()

        fire_blk(0, 0)
        fire_blk(1, 1)
        plsc.subcore_barrier()

        @pl.loop(0, NBLK // 2)
        def _pair(t):
            for half in range(2):
                bs = half
                wait_blk(bs)
                for jl in range(SUBS):
                    q = jl % 4
                    if half == 0 and jl < 4:
                        @pl.when(t > 0)
                        def _():
                            wait_scatter(q)
                    else:
                        wait_scatter(q)
                    if jl == 4:
                        if half == 0:
                            @pl.when(t > 0)
                            def _():
                                fire_blk(t * 2 + 1, 1)
                        else:
                            @pl.when(t < NBLK // 2 - 1)
                            def _():
                                fire_blk(t * 2 + 2, 0)
                    fire_gather(bs, jl, q)
                    p = (jl - 2) % 4
                    if jl < 2:
                        if half == 0:
                            @pl.when(t > 0)
                            def _():
                                wait_gather(p)
                                fire_scatter(1, SUBS - 2 + jl, p)
                        else:
                            wait_gather(p)
                            fire_scatter(0, SUBS - 2 + jl, p)
                    else:
                        wait_gather(p)
                        fire_scatter(bs, jl - 2, p)

        wait_gather(2)
        fire_scatter(1, SUBS - 2, 2)
        wait_gather(3)
        fire_scatter(1, SUBS - 1, 3)
        for q in range(4):
            wait_scatter(q)
        plsc.subcore_barrier()
        pltpu.sync_copy(acc.at[pl.ds(r0, RPS)],
                        part_hbm.at[pl.ds(c * NP + r0, RPS)])

    return k(y, src3, dst3, zeros2)


def _prep_kernel(xp, w1p, deg0, deg1):
    def body(x_ref, w_ref, d0_ref, d1_ref, y_ref, dinv_ref):
        deg = d0_ref[...] + d1_ref[...] + 1.0
        dinv = lax.rsqrt(deg)
        xw = jnp.dot(x_ref[...], w_ref[...],
                     preferred_element_type=jnp.float32)
        y_ref[...] = xw * dinv
        dinv_ref[...] = dinv

    return pl.pallas_call(
        body,
        grid=(GRID,),
        in_specs=[
            pl.BlockSpec((RB, IN_PAD), lambda i: (i, 0)),
            pl.BlockSpec((IN_PAD, H), lambda i: (0, 0)),
            pl.BlockSpec((RB, 1), lambda i: (i, 0)),
            pl.BlockSpec((RB, 1), lambda i: (i, 0)),
        ],
        out_specs=[
            pl.BlockSpec((RB, H), lambda i: (i, 0)),
            pl.BlockSpec((RB, 1), lambda i: (i, 0)),
        ],
        out_shape=[
            jax.ShapeDtypeStruct((NP, H), jnp.float32),
            jax.ShapeDtypeStruct((NP, 1), jnp.float32),
        ],
    )(xp, w1p, deg0, deg1)


def _combine_kernel(p0, p1, y, dinv, b, w_next):
    def body(p0_ref, p1_ref, y_ref, dinv_ref, b_ref, w_ref, out_ref):
        dinv = dinv_ref[...]
        h = (p0_ref[...] + p1_ref[...] + y_ref[...]) * dinv + b_ref[...]
        h = jnp.maximum(h, 0.0)
        out_ref[...] = jnp.dot(h, w_ref[...],
                               preferred_element_type=jnp.float32) * dinv

    return pl.pallas_call(
        body,
        grid=(GRID,),
        in_specs=[
            pl.BlockSpec((RB, H), lambda i: (i, 0)),
            pl.BlockSpec((RB, H), lambda i: (i, 0)),
            pl.BlockSpec((RB, H), lambda i: (i, 0)),
            pl.BlockSpec((RB, 1), lambda i: (i, 0)),
            pl.BlockSpec((1, H), lambda i: (0, 0)),
            pl.BlockSpec((H, H), lambda i: (0, 0)),
        ],
        out_specs=pl.BlockSpec((RB, H), lambda i: (i, 0)),
        out_shape=jax.ShapeDtypeStruct((NP, H), jnp.float32),
    )(p0, p1, y, dinv, b, w_next)


def _final_kernel(p0, p1, y, dinv, b3, q1, qb1, q2, qb2, q3, qb3):
    def body(p0_ref, p1_ref, y_ref, dinv_ref, b_ref,
             q1_ref, qb1_ref, q2_ref, qb2_ref, q3_ref, qb3_ref, out_ref):
        dinv = dinv_ref[...]
        h = (p0_ref[...] + p1_ref[...] + y_ref[...]) * dinv + b_ref[...]
        h = jnp.maximum(h, 0.0)
        t = jnp.dot(h, q1_ref[...], preferred_element_type=jnp.float32)
        t = t + qb1_ref[...]
        t = jnp.where(t >= 0.0, t, 0.01 * t)
        t = jnp.dot(t, q2_ref[...], preferred_element_type=jnp.float32)
        t = t + qb2_ref[...]
        t = jnp.where(t >= 0.0, t, 0.01 * t)
        t = jnp.dot(t, q3_ref[...], preferred_element_type=jnp.float32)
        out_ref[...] = t + qb3_ref[...]

    return pl.pallas_call(
        body,
        grid=(GRID,),
        in_specs=[
            pl.BlockSpec((RB, H), lambda i: (i, 0)),
            pl.BlockSpec((RB, H), lambda i: (i, 0)),
            pl.BlockSpec((RB, H), lambda i: (i, 0)),
            pl.BlockSpec((RB, 1), lambda i: (i, 0)),
            pl.BlockSpec((1, H), lambda i: (0, 0)),
            pl.BlockSpec((H, H), lambda i: (0, 0)),
            pl.BlockSpec((1, H), lambda i: (0, 0)),
            pl.BlockSpec((H, H), lambda i: (0, 0)),
            pl.BlockSpec((1, H), lambda i: (0, 0)),
            pl.BlockSpec((H, 1), lambda i: (0, 0)),
            pl.BlockSpec((1, 1), lambda i: (0, 0)),
        ],
        out_specs=pl.BlockSpec((RB, 1), lambda i: (i, 0)),
        out_shape=jax.ShapeDtypeStruct((NP, 1), jnp.float32),
    )(p0, p1, y, dinv, b3, q1, qb1, q2, qb2, q3, qb3)


def kernel(x, edge_index, W1, b1, W2, b2, W3, b3, Q1, qb1, Q2, qb2, Q3, qb3):
    pad = N + (jnp.arange(E2 - E, dtype=jnp.int32) % (NP - N))
    src3 = jnp.concatenate([edge_index[0], pad]).reshape(NW, CH, B)
    dst3 = jnp.concatenate([edge_index[1], pad]).reshape(NW, CH, B)

    xp = jnp.zeros((NP, IN_PAD), jnp.float32).at[:N, :x.shape[1]].set(x)
    w1p = jnp.zeros((IN_PAD, H), jnp.float32).at[:W1.shape[0], :].set(W1)
    zeros1 = jnp.zeros((NP,), jnp.float32)
    zeros2 = jnp.zeros((NP, H), jnp.float32)

    degp = _deg_kernel(dst3, zeros1)
    deg0 = degp[:NP].reshape(NP, 1)
    deg1 = degp[NP:].reshape(NP, 1)

    y1, dinv = _prep_kernel(xp, w1p, deg0, deg1)

    p = _agg_kernel(y1, src3, dst3, zeros2)
    y2 = _combine_kernel(p[:NP], p[NP:], y1, dinv, b1.reshape(1, H), W2)

    p = _agg_kernel(y2, src3, dst3, zeros2)
    y3 = _combine_kernel(p[:NP], p[NP:], y2, dinv, b2.reshape(1, H), W3)

    p = _agg_kernel(y3, src3, dst3, zeros2)
    out = _final_kernel(p[:NP], p[NP:], y3, dinv, b3.reshape(1, H),
                        Q1, qb1.reshape(1, H), Q2, qb2.reshape(1, H),
                        Q3, qb3.reshape(1, 1))
    return out[:N]


# speedup vs baseline: 4.6408x; 1.0615x over previous
import functools

import jax
import jax.numpy as jnp
from jax import lax
from jax.experimental import pallas as pl
from jax.experimental.pallas import tpu as pltpu
from jax.experimental.pallas import tpu_sc as plsc

N = 10000
E = 640000
H = 128
IN_PAD = 8

NC = 2
NS = 16
NW = NC * NS
NP = 10240
RPS = NP // NS

B = 128
CH = 160
E2 = NW * CH * B
IBR = 4
NBLK = CH // IBR
BSUB = 64
SUBS = IBR * 2
RING = 5

RB = 1024
GRID = NP // RB


def _sc_mesh():
    return plsc.VectorSubcoreMesh(core_axis_name="c", subcore_axis_name="s")


def _deg_kernel(dst3, zeros1):
    @functools.partial(
        pl.kernel,
        out_type=jax.ShapeDtypeStruct((NC * NP,), jnp.float32),
        mesh=_sc_mesh(),
        scratch_types=[
            pltpu.VMEM((CH, B), jnp.int32),
            pltpu.VMEM((B,), jnp.float32),
            pltpu.VMEM_SHARED((NP,), jnp.float32),
            pltpu.SemaphoreType.DMA,
        ],
    )
    def k(dst_hbm, z_hbm, degp_hbm, dst_all, ones_v, acc, dsem):
        c = lax.axis_index("c")
        s = lax.axis_index("s")
        wid = c * NS + s
        r0 = s * RPS
        for i in range(B // 16):
            ones_v[pl.ds(i * 16, 16)] = jnp.ones((16,), jnp.float32)
        pltpu.sync_copy(dst_hbm.at[wid], dst_all)
        pltpu.sync_copy(z_hbm.at[pl.ds(r0, RPS)], acc.at[pl.ds(r0, RPS)])
        plsc.subcore_barrier()

        @pl.loop(0, CH)
        def _fire(j):
            pltpu.async_copy(ones_v, acc.at[dst_all.at[j]], dsem, add=True)

        @pl.loop(0, CH)
        def _drain(j):
            pltpu.make_async_copy(ones_v, acc.at[dst_all.at[0]], dsem).wait()

        plsc.subcore_barrier()
        pltpu.sync_copy(acc.at[pl.ds(r0, RPS)],
                        degp_hbm.at[pl.ds(c * NP + r0, RPS)])

    return k(dst3, zeros1)


def _agg_kernel(y, src3, dst3, zeros2):
    @functools.partial(
        pl.kernel,
        out_type=jax.ShapeDtypeStruct((NC * NP, H), jnp.float32),
        mesh=_sc_mesh(),
        scratch_types=[
            pltpu.VMEM((RING, IBR, B), jnp.int32),
            pltpu.VMEM((RING, IBR, B), jnp.int32),
            pltpu.VMEM((RING, BSUB, H), jnp.float32),
            pltpu.VMEM_SHARED((NP, H), jnp.float32),
            pltpu.SemaphoreType.DMA((RING,)),
            pltpu.SemaphoreType.DMA((RING,)),
            pltpu.SemaphoreType.DMA((RING,)),
        ],
    )
    def k(y_hbm, src_hbm, dst_hbm, z_hbm, part_hbm,
          sblk, dblk, rows, acc, bsem, gsem, ssem):
        c = lax.axis_index("c")
        s = lax.axis_index("s")
        wid = c * NS + s
        r0 = s * RPS
        pltpu.sync_copy(z_hbm.at[pl.ds(r0, RPS)], acc.at[pl.ds(r0, RPS)])

        def fire_blk(kb, bs):
            pltpu.async_copy(src_hbm.at[wid].at[pl.ds(kb * IBR, IBR)],
                             sblk.at[bs], bsem.at[bs])
            pltpu.async_copy(dst_hbm.at[wid].at[pl.ds(kb * IBR, IBR)],
                             dblk.at[bs], bsem.at[bs])

        def wait_blk(bs):
            pltpu.make_async_copy(src_hbm.at[0].at[pl.ds(0, IBR)],
                                  sblk.at[bs], bsem.at[bs]).wait()
            pltpu.make_async_copy(dst_hbm.at[0].at[pl.ds(0, IBR)],
                                  dblk.at[bs], bsem.at[bs]).wait()

        def idx(blk, bs, jl):
            return blk.at[bs, jl // 2, pl.ds((jl % 2) * BSUB, BSUB)]

        def fire_gather(bs, jl, q):
            pltpu.async_copy(y_hbm.at[idx(sblk, bs, jl)], rows.at[q],
                             gsem.at[q])

        def wait_gather(q):
            pltpu.make_async_copy(y_hbm.at[idx(sblk, 0, 0)], rows.at[q],
                                  gsem.at[q]).wait()

        def fire_scatter(bs, jl, q):
            pltpu.async_copy(rows.at[q], acc.at[idx(dblk, bs, jl)],
                             ssem.at[q], add=True)

        def wait_scatter(q):
            pltpu.make_async_copy(rows.at[0], acc.at[idx(dblk, 0, 0)],
                                  ssem.at[q]).wait()

        for kb in range(RING - 1):
            fire_blk(kb, kb)
        plsc.subcore_barrier()

        @pl.loop(0, NBLK // RING)
        def _grp(t):
            for kk in range(RING):
                bs = kk
                wait_blk(bs)
                for jl in range(SUBS):
                    q = (kk * SUBS + jl) % RING
                    first = kk == 0 and jl < RING
                    if first:
                        @pl.when(t > 0)
                        def _():
                            wait_scatter(q)
                    else:
                        wait_scatter(q)
                    if jl == 4:
                        @pl.when(t * RING + kk + (RING - 1) < NBLK)
                        def _():
                            fire_blk(t * RING + kk + (RING - 1),
                                     (kk + RING - 1) % RING)
                    fire_gather(bs, jl, q)
                    p = (kk * SUBS + jl - 3) % RING
                    if jl < 3:
                        pbs = (kk - 1) % RING
                        pjl = SUBS - 3 + jl
                        if kk == 0:
                            @pl.when(t > 0)
                            def _():
                                wait_gather(p)
                                fire_scatter(pbs, pjl, p)
                        else:
                            wait_gather(p)
                            fire_scatter(pbs, pjl, p)
                    else:
                        wait_gather(p)
                        fire_scatter(bs, jl - 3, p)

        last = NBLK * SUBS
        lbs = (NBLK - 1) % RING
        for j in range(last - 3, last):
            wait_gather(j % RING)
            fire_scatter(lbs, j - (NBLK - 1) * SUBS, j % RING)
        for q in range(RING):
            wait_scatter(q)
        plsc.subcore_barrier()
        pltpu.sync_copy(acc.at[pl.ds(r0, RPS)],
                        part_hbm.at[pl.ds(c * NP + r0, RPS)])

    return k(y, src3, dst3, zeros2)


def _prep_kernel(xp, w1p, deg0, deg1):
    def body(x_ref, w_ref, d0_ref, d1_ref, y_ref, dinv_ref):
        deg = d0_ref[...] + d1_ref[...] + 1.0
        dinv = lax.rsqrt(deg)
        xw = jnp.dot(x_ref[...], w_ref[...],
                     preferred_element_type=jnp.float32)
        y_ref[...] = xw * dinv
        dinv_ref[...] = dinv

    return pl.pallas_call(
        body,
        grid=(GRID,),
        in_specs=[
            pl.BlockSpec((RB, IN_PAD), lambda i: (i, 0)),
            pl.BlockSpec((IN_PAD, H), lambda i: (0, 0)),
            pl.BlockSpec((RB, 1), lambda i: (i, 0)),
            pl.BlockSpec((RB, 1), lambda i: (i, 0)),
        ],
        out_specs=[
            pl.BlockSpec((RB, H), lambda i: (i, 0)),
            pl.BlockSpec((RB, 1), lambda i: (i, 0)),
        ],
        out_shape=[
            jax.ShapeDtypeStruct((NP, H), jnp.float32),
            jax.ShapeDtypeStruct((NP, 1), jnp.float32),
        ],
    )(xp, w1p, deg0, deg1)


def _combine_kernel(p0, p1, y, dinv, b, w_next):
    def body(p0_ref, p1_ref, y_ref, dinv_ref, b_ref, w_ref, out_ref):
        dinv = dinv_ref[...]
        h = (p0_ref[...] + p1_ref[...] + y_ref[...]) * dinv + b_ref[...]
        h = jnp.maximum(h, 0.0)
        out_ref[...] = jnp.dot(h, w_ref[...],
                               preferred_element_type=jnp.float32) * dinv

    return pl.pallas_call(
        body,
        grid=(GRID,),
        in_specs=[
            pl.BlockSpec((RB, H), lambda i: (i, 0)),
            pl.BlockSpec((RB, H), lambda i: (i, 0)),
            pl.BlockSpec((RB, H), lambda i: (i, 0)),
            pl.BlockSpec((RB, 1), lambda i: (i, 0)),
            pl.BlockSpec((1, H), lambda i: (0, 0)),
            pl.BlockSpec((H, H), lambda i: (0, 0)),
        ],
        out_specs=pl.BlockSpec((RB, H), lambda i: (i, 0)),
        out_shape=jax.ShapeDtypeStruct((NP, H), jnp.float32),
    )(p0, p1, y, dinv, b, w_next)


def _final_kernel(p0, p1, y, dinv, b3, q1, qb1, q2, qb2, q3, qb3):
    def body(p0_ref, p1_ref, y_ref, dinv_ref, b_ref,
             q1_ref, qb1_ref, q2_ref, qb2_ref, q3_ref, qb3_ref, out_ref):
        dinv = dinv_ref[...]
        h = (p0_ref[...] + p1_ref[...] + y_ref[...]) * dinv + b_ref[...]
        h = jnp.maximum(h, 0.0)
        t = jnp.dot(h, q1_ref[...], preferred_element_type=jnp.float32)
        t = t + qb1_ref[...]
        t = jnp.where(t >= 0.0, t, 0.01 * t)
        t = jnp.dot(t, q2_ref[...], preferred_element_type=jnp.float32)
        t = t + qb2_ref[...]
        t = jnp.where(t >= 0.0, t, 0.01 * t)
        t = jnp.dot(t, q3_ref[...], preferred_element_type=jnp.float32)
        out_ref[...] = t + qb3_ref[...]

    return pl.pallas_call(
        body,
        grid=(GRID,),
        in_specs=[
            pl.BlockSpec((RB, H), lambda i: (i, 0)),
            pl.BlockSpec((RB, H), lambda i: (i, 0)),
            pl.BlockSpec((RB, H), lambda i: (i, 0)),
            pl.BlockSpec((RB, 1), lambda i: (i, 0)),
            pl.BlockSpec((1, H), lambda i: (0, 0)),
            pl.BlockSpec((H, H), lambda i: (0, 0)),
            pl.BlockSpec((1, H), lambda i: (0, 0)),
            pl.BlockSpec((H, H), lambda i: (0, 0)),
            pl.BlockSpec((1, H), lambda i: (0, 0)),
            pl.BlockSpec((H, 1), lambda i: (0, 0)),
            pl.BlockSpec((1, 1), lambda i: (0, 0)),
        ],
        out_specs=pl.BlockSpec((RB, 1), lambda i: (i, 0)),
        out_shape=jax.ShapeDtypeStruct((NP, 1), jnp.float32),
    )(p0, p1, y, dinv, b3, q1, qb1, q2, qb2, q3, qb3)


def kernel(x, edge_index, W1, b1, W2, b2, W3, b3, Q1, qb1, Q2, qb2, Q3, qb3):
    pad = N + (jnp.arange(E2 - E, dtype=jnp.int32) % (NP - N))
    src3 = jnp.concatenate([edge_index[0], pad]).reshape(NW, CH, B)
    dst3 = jnp.concatenate([edge_index[1], pad]).reshape(NW, CH, B)

    xp = jnp.zeros((NP, IN_PAD), jnp.float32).at[:N, :x.shape[1]].set(x)
    w1p = jnp.zeros((IN_PAD, H), jnp.float32).at[:W1.shape[0], :].set(W1)
    zeros1 = jnp.zeros((NP,), jnp.float32)
    zeros2 = jnp.zeros((NP, H), jnp.float32)

    degp = _deg_kernel(dst3, zeros1)
    deg0 = degp[:NP].reshape(NP, 1)
    deg1 = degp[NP:].reshape(NP, 1)

    y1, dinv = _prep_kernel(xp, w1p, deg0, deg1)

    p = _agg_kernel(y1, src3, dst3, zeros2)
    y2 = _combine_kernel(p[:NP], p[NP:], y1, dinv, b1.reshape(1, H), W2)

    p = _agg_kernel(y2, src3, dst3, zeros2)
    y3 = _combine_kernel(p[:NP], p[NP:], y2, dinv, b2.reshape(1, H), W3)

    p = _agg_kernel(y3, src3, dst3, zeros2)
    out = _final_kernel(p[:NP], p[NP:], y3, dinv, b3.reshape(1, H),
                        Q1, qb1.reshape(1, H), Q2, qb2.reshape(1, H),
                        Q3, qb3.reshape(1, 1))
    return out[:N]


# speedup vs baseline: 4.6449x; 1.0009x over previous
import functools

import jax
import jax.numpy as jnp
from jax import lax
from jax.experimental import pallas as pl
from jax.experimental.pallas import tpu as pltpu
from jax.experimental.pallas import tpu_sc as plsc

N = 10000
E = 640000
H = 128
IN_PAD = 8

NC = 2
NS = 16
NW = NC * NS
NP = 10240
RPS = NP // NS

B = 128
CH = 160
E2 = NW * CH * B
IBR = 4
NBLK = CH // IBR
BSUB = 64
SUBS = IBR * 2
RING = 5

RB = 1024
GRID = NP // RB


def _sc_mesh():
    return plsc.VectorSubcoreMesh(core_axis_name="c", subcore_axis_name="s")


def _deg_kernel(dst3, zeros1):
    @functools.partial(
        pl.kernel,
        out_type=jax.ShapeDtypeStruct((NC * NP,), jnp.float32),
        mesh=_sc_mesh(),
        scratch_types=[
            pltpu.VMEM((CH, B), jnp.int32),
            pltpu.VMEM((B,), jnp.float32),
            pltpu.VMEM_SHARED((NP,), jnp.float32),
            pltpu.SemaphoreType.DMA,
        ],
    )
    def k(dst_hbm, z_hbm, degp_hbm, dst_all, ones_v, acc, dsem):
        c = lax.axis_index("c")
        s = lax.axis_index("s")
        wid = c * NS + s
        r0 = s * RPS
        for i in range(B // 16):
            ones_v[pl.ds(i * 16, 16)] = jnp.ones((16,), jnp.float32)
        pltpu.sync_copy(dst_hbm.at[wid], dst_all)
        pltpu.sync_copy(z_hbm.at[pl.ds(r0, RPS)], acc.at[pl.ds(r0, RPS)])
        plsc.subcore_barrier()

        @pl.loop(0, CH)
        def _fire(j):
            pltpu.async_copy(ones_v, acc.at[dst_all.at[j]], dsem, add=True)

        @pl.loop(0, CH)
        def _drain(j):
            pltpu.make_async_copy(ones_v, acc.at[dst_all.at[0]], dsem).wait()

        plsc.subcore_barrier()
        pltpu.sync_copy(acc.at[pl.ds(r0, RPS)],
                        degp_hbm.at[pl.ds(c * NP + r0, RPS)])

    return k(dst3, zeros1)


def _agg_kernel(y, src3, dst3, zeros2):
    @functools.partial(
        pl.kernel,
        out_type=jax.ShapeDtypeStruct((NC * NP, H), jnp.float32),
        mesh=_sc_mesh(),
        scratch_types=[
            pltpu.VMEM((RING, IBR, B), jnp.int32),
            pltpu.VMEM((RING, IBR, B), jnp.int32),
            pltpu.VMEM((RING, BSUB, H), jnp.float32),
            pltpu.VMEM_SHARED((NP, H), jnp.float32),
            pltpu.SemaphoreType.DMA((RING,)),
            pltpu.SemaphoreType.DMA((RING,)),
            pltpu.SemaphoreType.DMA((RING,)),
        ],
    )
    def k(y_hbm, src_hbm, dst_hbm, z_hbm, part_hbm,
          sblk, dblk, rows, acc, bsem, gsem, ssem):
        c = lax.axis_index("c")
        s = lax.axis_index("s")
        wid = c * NS + s
        r0 = s * RPS

        def fire_blk(kb, bs):
            pltpu.async_copy(src_hbm.at[wid].at[pl.ds(kb * IBR, IBR)],
                             sblk.at[bs], bsem.at[bs])
            pltpu.async_copy(dst_hbm.at[wid].at[pl.ds(kb * IBR, IBR)],
                             dblk.at[bs], bsem.at[bs])

        def wait_blk(bs):
            pltpu.make_async_copy(src_hbm.at[0].at[pl.ds(0, IBR)],
                                  sblk.at[bs], bsem.at[bs]).wait()
            pltpu.make_async_copy(dst_hbm.at[0].at[pl.ds(0, IBR)],
                                  dblk.at[bs], bsem.at[bs]).wait()

        def idx(blk, bs, jl):
            return blk.at[bs, jl // 2, pl.ds((jl % 2) * BSUB, BSUB)]

        def fire_gather(bs, jl, q):
            pltpu.async_copy(y_hbm.at[idx(sblk, bs, jl)], rows.at[q],
                             gsem.at[q])

        def wait_gather(q):
            pltpu.make_async_copy(y_hbm.at[idx(sblk, 0, 0)], rows.at[q],
                                  gsem.at[q]).wait()

        def fire_scatter(bs, jl, q):
            pltpu.async_copy(rows.at[q], acc.at[idx(dblk, bs, jl)],
                             ssem.at[q], add=True)

        def wait_scatter(q):
            pltpu.make_async_copy(rows.at[0], acc.at[idx(dblk, 0, 0)],
                                  ssem.at[q]).wait()

        for kb in range(RING - 1):
            fire_blk(kb, kb)
        pltpu.sync_copy(z_hbm.at[pl.ds(r0, RPS)], acc.at[pl.ds(r0, RPS)])
        plsc.subcore_barrier()

        @pl.loop(0, NBLK // RING)
        def _grp(t):
            for kk in range(RING):
                bs = kk
                wait_blk(bs)
                for jl in range(SUBS):
                    q = (kk * SUBS + jl) % RING
                    first = kk == 0 and jl < RING
                    if first:
                        @pl.when(t > 0)
                        def _():
                            wait_scatter(q)
                    else:
                        wait_scatter(q)
                    if jl == 4:
                        @pl.when(t * RING + kk + (RING - 1) < NBLK)
                        def _():
                            fire_blk(t * RING + kk + (RING - 1),
                                     (kk + RING - 1) % RING)
                    fire_gather(bs, jl, q)
                    p = (kk * SUBS + jl - 3) % RING
                    if jl < 3:
                        pbs = (kk - 1) % RING
                        pjl = SUBS - 3 + jl
                        if kk == 0:
                            @pl.when(t > 0)
                            def _():
                                wait_gather(p)
                                fire_scatter(pbs, pjl, p)
                        else:
                            wait_gather(p)
                            fire_scatter(pbs, pjl, p)
                    else:
                        wait_gather(p)
                        fire_scatter(bs, jl - 3, p)

        last = NBLK * SUBS
        lbs = (NBLK - 1) % RING
        for j in range(last - 3, last):
            wait_gather(j % RING)
            fire_scatter(lbs, j - (NBLK - 1) * SUBS, j % RING)
        for q in range(RING):
            wait_scatter(q)
        plsc.subcore_barrier()
        pltpu.sync_copy(acc.at[pl.ds(r0, RPS)],
                        part_hbm.at[pl.ds(c * NP + r0, RPS)])

    return k(y, src3, dst3, zeros2)


def _prep_kernel(xp, w1p, deg0, deg1):
    def body(x_ref, w_ref, d0_ref, d1_ref, y_ref, dinv_ref):
        deg = d0_ref[...] + d1_ref[...] + 1.0
        dinv = lax.rsqrt(deg)
        xw = jnp.dot(x_ref[...], w_ref[...],
                     preferred_element_type=jnp.float32)
        y_ref[...] = xw * dinv
        dinv_ref[...] = dinv

    return pl.pallas_call(
        body,
        grid=(GRID,),
        in_specs=[
            pl.BlockSpec((RB, IN_PAD), lambda i: (i, 0)),
            pl.BlockSpec((IN_PAD, H), lambda i: (0, 0)),
            pl.BlockSpec((RB, 1), lambda i: (i, 0)),
            pl.BlockSpec((RB, 1), lambda i: (i, 0)),
        ],
        out_specs=[
            pl.BlockSpec((RB, H), lambda i: (i, 0)),
            pl.BlockSpec((RB, 1), lambda i: (i, 0)),
        ],
        out_shape=[
            jax.ShapeDtypeStruct((NP, H), jnp.float32),
            jax.ShapeDtypeStruct((NP, 1), jnp.float32),
        ],
    )(xp, w1p, deg0, deg1)


def _combine_kernel(p0, p1, y, dinv, b, w_next):
    def body(p0_ref, p1_ref, y_ref, dinv_ref, b_ref, w_ref, out_ref):
        dinv = dinv_ref[...]
        h = (p0_ref[...] + p1_ref[...] + y_ref[...]) * dinv + b_ref[...]
        h = jnp.maximum(h, 0.0)
        out_ref[...] = jnp.dot(h, w_ref[...],
                               preferred_element_type=jnp.float32) * dinv

    return pl.pallas_call(
        body,
        grid=(GRID,),
        in_specs=[
            pl.BlockSpec((RB, H), lambda i: (i, 0)),
            pl.BlockSpec((RB, H), lambda i: (i, 0)),
            pl.BlockSpec((RB, H), lambda i: (i, 0)),
            pl.BlockSpec((RB, 1), lambda i: (i, 0)),
            pl.BlockSpec((1, H), lambda i: (0, 0)),
            pl.BlockSpec((H, H), lambda i: (0, 0)),
        ],
        out_specs=pl.BlockSpec((RB, H), lambda i: (i, 0)),
        out_shape=jax.ShapeDtypeStruct((NP, H), jnp.float32),
    )(p0, p1, y, dinv, b, w_next)


def _final_kernel(p0, p1, y, dinv, b3, q1, qb1, q2, qb2, q3, qb3):
    def body(p0_ref, p1_ref, y_ref, dinv_ref, b_ref,
             q1_ref, qb1_ref, q2_ref, qb2_ref, q3_ref, qb3_ref, out_ref):
        dinv = dinv_ref[...]
        h = (p0_ref[...] + p1_ref[...] + y_ref[...]) * dinv + b_ref[...]
        h = jnp.maximum(h, 0.0)
        t = jnp.dot(h, q1_ref[...], preferred_element_type=jnp.float32)
        t = t + qb1_ref[...]
        t = jnp.where(t >= 0.0, t, 0.01 * t)
        t = jnp.dot(t, q2_ref[...], preferred_element_type=jnp.float32)
        t = t + qb2_ref[...]
        t = jnp.where(t >= 0.0, t, 0.01 * t)
        t = jnp.dot(t, q3_ref[...], preferred_element_type=jnp.float32)
        out_ref[...] = t + qb3_ref[...]

    return pl.pallas_call(
        body,
        grid=(GRID,),
        in_specs=[
            pl.BlockSpec((RB, H), lambda i: (i, 0)),
            pl.BlockSpec((RB, H), lambda i: (i, 0)),
            pl.BlockSpec((RB, H), lambda i: (i, 0)),
            pl.BlockSpec((RB, 1), lambda i: (i, 0)),
            pl.BlockSpec((1, H), lambda i: (0, 0)),
            pl.BlockSpec((H, H), lambda i: (0, 0)),
            pl.BlockSpec((1, H), lambda i: (0, 0)),
            pl.BlockSpec((H, H), lambda i: (0, 0)),
            pl.BlockSpec((1, H), lambda i: (0, 0)),
            pl.BlockSpec((H, 1), lambda i: (0, 0)),
            pl.BlockSpec((1, 1), lambda i: (0, 0)),
        ],
        out_specs=pl.BlockSpec((RB, 1), lambda i: (i, 0)),
        out_shape=jax.ShapeDtypeStruct((NP, 1), jnp.float32),
    )(p0, p1, y, dinv, b3, q1, qb1, q2, qb2, q3, qb3)


def kernel(x, edge_index, W1, b1, W2, b2, W3, b3, Q1, qb1, Q2, qb2, Q3, qb3):
    pad = N + (jnp.arange(E2 - E, dtype=jnp.int32) % (NP - N))
    src3 = jnp.concatenate([edge_index[0], pad]).reshape(NW, CH, B)
    dst3 = jnp.concatenate([edge_index[1], pad]).reshape(NW, CH, B)

    xp = jnp.zeros((NP, IN_PAD), jnp.float32).at[:N, :x.shape[1]].set(x)
    w1p = jnp.zeros((IN_PAD, H), jnp.float32).at[:W1.shape[0], :].set(W1)
    zeros1 = jnp.zeros((NP,), jnp.float32)
    zeros2 = jnp.zeros((NP, H), jnp.float32)

    degp = _deg_kernel(dst3, zeros1)
    deg0 = degp[:NP].reshape(NP, 1)
    deg1 = degp[NP:].reshape(NP, 1)

    y1, dinv = _prep_kernel(xp, w1p, deg0, deg1)

    p = _agg_kernel(y1, src3, dst3, zeros2)
    y2 = _combine_kernel(p[:NP], p[NP:], y1, dinv, b1.reshape(1, H), W2)

    p = _agg_kernel(y2, src3, dst3, zeros2)
    y3 = _combine_kernel(p[:NP], p[NP:], y2, dinv, b2.reshape(1, H), W3)

    p = _agg_kernel(y3, src3, dst3, zeros2)
    out = _final_kernel(p[:NP], p[NP:], y3, dinv, b3.reshape(1, H),
                        Q1, qb1.reshape(1, H), Q2, qb2.reshape(1, H),
                        Q3, qb3.reshape(1, 1))
    return out[:N]
